# baseline (speedup 1.0000x reference)
import functools

import jax
import jax.numpy as jnp
from jax.experimental import pallas as pl
from jax.experimental.pallas import tpu as pltpu

_NUM_PERS = 4
_D = 64
_TOPK = 16
_NEG = -3.0


def _normalize_feats(x, w, scale=1.0):
    sq = jax.lax.dot_general(
        x * x, w * w,
        dimension_numbers=(((1,), (1,)), ((), ())),
        precision=jax.lax.Precision.HIGHEST,
        preferred_element_type=jnp.float32,
    )
    inv = scale / jnp.maximum(jnp.sqrt(sq), 1e-12)
    feats = []
    for p in range(_NUM_PERS):
        feats.append((x * w[p][None, :] * inv[:, p:p + 1]).astype(jnp.bfloat16))
    return jnp.concatenate(feats, axis=1)


_CHUNK = 16


def _block_kernel(nodes_ref, pivots_ref, w_ref, out_ref, pfeat_ref, scores_ref):
    pid = pl.program_id(0)

    @pl.when(pid == 0)
    def _():
        pfeat_ref[...] = _normalize_feats(pivots_ref[...], w_ref[...], scale=0.25)

    nfeat = _normalize_feats(nodes_ref[...], w_ref[...])
    scores_ref[...] = jax.lax.dot_general(
        nfeat, pfeat_ref[...],
        dimension_numbers=(((1,), (1,)), ((), ())),
        preferred_element_type=jnp.float32,
    )

    bn = out_ref.shape[0]

    def body(i, carry):
        a = scores_ref[pl.ds(i * _CHUNK, _CHUNK), :]
        b = a
        for _ in range(_TOPK):
            m = jnp.max(b, axis=1, keepdims=True)
            b = jnp.where(b == m, _NEG, b)
        out_ref[pl.ds(i * _CHUNK, _CHUNK), :] = jnp.where(b == _NEG, a, 0.0)
        return carry

    jax.lax.fori_loop(0, bn // _CHUNK, body, 0)


@jax.jit
def kernel(nodes, pivots, weight_tensor):
    n, d = nodes.shape
    m = pivots.shape[0]
    bn = 400
    grid = n // bn
    return pl.pallas_call(
        _block_kernel,
        grid=(grid,),
        in_specs=[
            pl.BlockSpec((bn, d), lambda i: (i, 0)),
            pl.BlockSpec((m, d), lambda i: (0, 0)),
            pl.BlockSpec((_NUM_PERS, d), lambda i: (0, 0)),
        ],
        out_specs=pl.BlockSpec((bn, m), lambda i: (i, 0)),
        out_shape=jax.ShapeDtypeStruct((n, m), jnp.float32),
        scratch_shapes=[
            pltpu.VMEM((m, _NUM_PERS * d), jnp.bfloat16),
            pltpu.VMEM((bn, m), jnp.float32),
        ],
    )(nodes, pivots, weight_tensor)


# speedup vs baseline: 8.4870x; 8.4870x over previous
import functools

import jax
import jax.numpy as jnp
from jax.experimental import pallas as pl
from jax.experimental.pallas import tpu as pltpu

_NUM_PERS = 4
_D = 64
_TOPK = 16
_NEG = -3.0


def _normalize_feats(x, w, scale=1.0):
    sq = jax.lax.dot_general(
        x * x, w * w,
        dimension_numbers=(((1,), (1,)), ((), ())),
        precision=jax.lax.Precision.HIGHEST,
        preferred_element_type=jnp.float32,
    )
    inv = scale / jnp.maximum(jnp.sqrt(sq), 1e-12)
    feats = []
    for p in range(_NUM_PERS):
        feats.append((x * w[p][None, :] * inv[:, p:p + 1]).astype(jnp.bfloat16))
    return jnp.concatenate(feats, axis=1)


_CHUNK = 16


def _block_kernel(nodes_ref, pivots_ref, w_ref, out_ref, pfeat_ref, scores_ref):
    pid = pl.program_id(0)

    @pl.when(pid == 0)
    def _():
        pfeat_ref[...] = _normalize_feats(pivots_ref[...], w_ref[...], scale=0.25)

    nfeat = _normalize_feats(nodes_ref[...], w_ref[...])
    scores_ref[...] = jax.lax.dot_general(
        nfeat, pfeat_ref[...],
        dimension_numbers=(((1,), (1,)), ((), ())),
        preferred_element_type=jnp.float32,
    )

    bn = out_ref.shape[0]

    for i in range(bn // _CHUNK):
        b = scores_ref[pl.ds(i * _CHUNK, _CHUNK), :]
        for _ in range(_TOPK):
            m = jnp.max(b, axis=1, keepdims=True)
            b = jnp.where(b == m, _NEG, b)
        sel = b == _NEG
        out_ref[pl.ds(i * _CHUNK, _CHUNK), :] = jnp.where(
            sel, scores_ref[pl.ds(i * _CHUNK, _CHUNK), :], 0.0)


@jax.jit
def kernel(nodes, pivots, weight_tensor):
    n, d = nodes.shape
    m = pivots.shape[0]
    bn = 400
    grid = n // bn
    return pl.pallas_call(
        _block_kernel,
        grid=(grid,),
        in_specs=[
            pl.BlockSpec((bn, d), lambda i: (i, 0)),
            pl.BlockSpec((m, d), lambda i: (0, 0)),
            pl.BlockSpec((_NUM_PERS, d), lambda i: (0, 0)),
        ],
        out_specs=pl.BlockSpec((bn, m), lambda i: (i, 0)),
        out_shape=jax.ShapeDtypeStruct((n, m), jnp.float32),
        scratch_shapes=[
            pltpu.VMEM((m, _NUM_PERS * d), jnp.bfloat16),
            pltpu.VMEM((bn, m), jnp.float32),
        ],
    )(nodes, pivots, weight_tensor)


# speedup vs baseline: 10.4337x; 1.2294x over previous
import functools

import jax
import jax.numpy as jnp
from jax.experimental import pallas as pl
from jax.experimental.pallas import tpu as pltpu

_NUM_PERS = 4
_D = 64
_TOPK = 16
_NEG = -3.0


def _normalize_feats(x, w, scale=1.0):
    sq = jax.lax.dot_general(
        x * x, w * w,
        dimension_numbers=(((1,), (1,)), ((), ())),
        precision=jax.lax.Precision.HIGHEST,
        preferred_element_type=jnp.float32,
    )
    inv = scale / jnp.maximum(jnp.sqrt(sq), 1e-12)
    feats = []
    for p in range(_NUM_PERS):
        feats.append((x * w[p][None, :] * inv[:, p:p + 1]).astype(jnp.bfloat16))
    return jnp.concatenate(feats, axis=1)


_CHUNK = 16


def _block_kernel(nodes_ref, pivots_ref, w_ref, out_ref, pfeat_ref, scores_ref):
    pid = pl.program_id(0)

    @pl.when(pid == 0)
    def _():
        pfeat_ref[...] = _normalize_feats(pivots_ref[...], w_ref[...], scale=0.25)

    nfeat = _normalize_feats(nodes_ref[...], w_ref[...])
    scores_ref[...] = jax.lax.dot_general(
        nfeat, pfeat_ref[...],
        dimension_numbers=(((1,), (1,)), ((), ())),
        preferred_element_type=jnp.float32,
    )

    bn = out_ref.shape[0]

    half = out_ref.shape[1] // 2
    for i in range(bn // _CHUNK):
        x = scores_ref[pl.ds(i * _CHUNK, _CHUNK), :]
        hi = x[:, :half]
        lo = x[:, half:]
        cmax = jnp.maximum(hi, lo)
        cmin = jnp.minimum(hi, lo)
        for _ in range(_TOPK):
            m = jnp.max(cmax, axis=1, keepdims=True)
            eq = cmax == m
            cmax = jnp.where(eq, cmin, cmax)
            cmin = jnp.where(eq, _NEG, cmin)
        x2 = scores_ref[pl.ds(i * _CHUNK, _CHUNK), :]
        out_ref[pl.ds(i * _CHUNK, _CHUNK), :] = jnp.where(x2 >= m, x2, 0.0)


@jax.jit
def kernel(nodes, pivots, weight_tensor):
    n, d = nodes.shape
    m = pivots.shape[0]
    bn = 400
    grid = n // bn
    return pl.pallas_call(
        _block_kernel,
        grid=(grid,),
        in_specs=[
            pl.BlockSpec((bn, d), lambda i: (i, 0)),
            pl.BlockSpec((m, d), lambda i: (0, 0)),
            pl.BlockSpec((_NUM_PERS, d), lambda i: (0, 0)),
        ],
        out_specs=pl.BlockSpec((bn, m), lambda i: (i, 0)),
        out_shape=jax.ShapeDtypeStruct((n, m), jnp.float32),
        scratch_shapes=[
            pltpu.VMEM((m, _NUM_PERS * d), jnp.bfloat16),
            pltpu.VMEM((bn, m), jnp.float32),
        ],
    )(nodes, pivots, weight_tensor)


# speedup vs baseline: 11.2471x; 1.0780x over previous
import functools

import jax
import jax.numpy as jnp
from jax.experimental import pallas as pl
from jax.experimental.pallas import tpu as pltpu

_NUM_PERS = 4
_D = 64
_TOPK = 16
_NEG = -3.0


def _normalize_feats(x, w, scale=1.0):
    sq = jax.lax.dot_general(
        x * x, w * w,
        dimension_numbers=(((1,), (1,)), ((), ())),
        precision=jax.lax.Precision.HIGHEST,
        preferred_element_type=jnp.float32,
    )
    inv = scale / jnp.maximum(jnp.sqrt(sq), 1e-12)
    feats = []
    for p in range(_NUM_PERS):
        feats.append((x * w[p][None, :] * inv[:, p:p + 1]).astype(jnp.bfloat16))
    return jnp.concatenate(feats, axis=1)


_CHUNK = 16


def _block_kernel(nodes_ref, pivots_ref, w_ref, out_ref, pfeat_ref, scores_ref):
    pid = pl.program_id(0)

    @pl.when(pid == 0)
    def _():
        pfeat_ref[...] = _normalize_feats(pivots_ref[...], w_ref[...], scale=0.25)

    nfeat = _normalize_feats(nodes_ref[...], w_ref[...])
    scores_ref[...] = jax.lax.dot_general(
        nfeat, pfeat_ref[...],
        dimension_numbers=(((1,), (1,)), ((), ())),
        preferred_element_type=jnp.float32,
    )

    bn = out_ref.shape[0]

    quart = out_ref.shape[1] // 4
    for i in range(bn // _CHUNK):
        x = scores_ref[pl.ds(i * _CHUNK, _CHUNK), :]
        s0 = x[:, :quart]
        s1 = x[:, quart:2 * quart]
        s2 = x[:, 2 * quart:3 * quart]
        s3 = x[:, 3 * quart:]
        s0, s1 = jnp.maximum(s0, s1), jnp.minimum(s0, s1)
        s2, s3 = jnp.maximum(s2, s3), jnp.minimum(s2, s3)
        s0, s2 = jnp.maximum(s0, s2), jnp.minimum(s0, s2)
        s1, s3 = jnp.maximum(s1, s3), jnp.minimum(s1, s3)
        s1, s2 = jnp.maximum(s1, s2), jnp.minimum(s1, s2)
        for _ in range(_TOPK):
            m = jnp.max(s0, axis=1, keepdims=True)
            eq = s0 == m
            s0 = jnp.where(eq, s1, s0)
            s1 = jnp.where(eq, s2, s1)
            s2 = jnp.where(eq, s3, s2)
            s3 = jnp.where(eq, _NEG, s3)
        x2 = scores_ref[pl.ds(i * _CHUNK, _CHUNK), :]
        out_ref[pl.ds(i * _CHUNK, _CHUNK), :] = jnp.where(x2 >= m, x2, 0.0)


@jax.jit
def kernel(nodes, pivots, weight_tensor):
    n, d = nodes.shape
    m = pivots.shape[0]
    bn = 400
    grid = n // bn
    return pl.pallas_call(
        _block_kernel,
        grid=(grid,),
        in_specs=[
            pl.BlockSpec((bn, d), lambda i: (i, 0)),
            pl.BlockSpec((m, d), lambda i: (0, 0)),
            pl.BlockSpec((_NUM_PERS, d), lambda i: (0, 0)),
        ],
        out_specs=pl.BlockSpec((bn, m), lambda i: (i, 0)),
        out_shape=jax.ShapeDtypeStruct((n, m), jnp.float32),
        scratch_shapes=[
            pltpu.VMEM((m, _NUM_PERS * d), jnp.bfloat16),
            pltpu.VMEM((bn, m), jnp.float32),
        ],
    )(nodes, pivots, weight_tensor)
